# baseline (speedup 1.0000x reference)
import functools

import jax
import jax.numpy as jnp
from jax import lax
from jax.experimental import pallas as pl
from jax.experimental.pallas import tpu as pltpu

HI = lax.Precision.HIGHEST
F32 = jnp.float32


def _amax_body(t_ref, am_ref):
    t = t_ref[...]
    B, L = t.shape
    mx = jnp.max(t, axis=1, keepdims=True)
    col = lax.broadcasted_iota(jnp.int32, (B, L), 1)
    cand = jnp.where(t == mx, col, L)
    am = jnp.min(cand, axis=1, keepdims=True)
    am_ref[...] = jnp.broadcast_to(am, (B, 8))


def _select_body(am_ref, t_ref, a_ref, f_ref, feats_ref, gid_ref, *, L, KP, K):
    b = pl.program_id(0)
    amb = am_ref[b]
    t = t_ref[0]
    a = a_ref[0]
    col1 = lax.broadcasted_iota(jnp.int32, (1, L), 1)
    a = jnp.where(col1 == amb, jnp.float32(-1.0), a)
    a = jnp.where(t != 0, a, jnp.float32(0.0))

    A = jnp.broadcast_to(a, (L, L))
    AT = lax.transpose(A, (1, 0))
    J = lax.broadcasted_iota(jnp.int32, (L, L), 1)
    I = lax.broadcasted_iota(jnp.int32, (L, L), 0)
    beat = (A > AT) | ((A == AT) & (J < I))
    rank = jnp.sum(beat.astype(F32), axis=1, keepdims=True)

    ranki = rank.astype(jnp.int32)
    R = lax.broadcasted_iota(jnp.int32, (L, KP), 1)
    O = (ranki == R).astype(F32)
    OT = lax.transpose(O, (1, 0))
    feats = lax.dot_general(OT, f_ref[0], (((1,), (0,)), ((), ())),
                            precision=HI, preferred_element_type=F32)
    feats_ref[0] = feats[:K, :]

    Irow = lax.broadcasted_iota(jnp.int32, (L, KP), 0).astype(F32)
    idxf = jnp.sum(O * Irow, axis=0, keepdims=True)
    gid_ref[0] = idxf.astype(jnp.int32) + b * L


def _stats_body(x_ref, w1t_ref, b1_ref, st_ref):
    i = pl.program_id(0)

    @pl.when(i == 0)
    def _():
        st_ref[...] = jnp.zeros_like(st_ref)

    x = x_ref[...]
    nrm = jnp.sqrt(jnp.sum(x * x, axis=1, keepdims=True)) + 1e-8
    xn = x / nrm
    h = lax.dot_general(xn, w1t_ref[...], (((1,), (0,)), ((), ())),
                        precision=HI, preferred_element_type=F32)
    h = h + b1_ref[...]
    s = jnp.sum(h, axis=0, keepdims=True)
    ss = jnp.sum(h * h, axis=0, keepdims=True)
    st_ref[0:1, :] += s
    st_ref[1:2, :] += ss


def _final_body(x_ref, w1t_ref, b1_ref, st_ref, g_ref, bb_ref, w2t_ref,
                b2_ref, lwt_ref, lb_ref, out_ref, *, n_rows):
    x = x_ref[...]
    nrm = jnp.sqrt(jnp.sum(x * x, axis=1, keepdims=True)) + 1e-8
    xn = x / nrm
    h = lax.dot_general(xn, w1t_ref[...], (((1,), (0,)), ((), ())),
                        precision=HI, preferred_element_type=F32)
    h = h + b1_ref[...]
    inv_n = jnp.float32(1.0 / n_rows)
    mean = st_ref[0:1, :] * inv_n
    var = st_ref[1:2, :] * inv_n - mean * mean
    hn = (h - mean) / jnp.sqrt(var + 1e-5) * g_ref[...] + bb_ref[...]
    hr = jnp.maximum(hn, 0.0)
    out = lax.dot_general(hr, w2t_ref[...], (((1,), (0,)), ((), ())),
                          precision=HI, preferred_element_type=F32)
    out = out + b2_ref[...]
    c = xn * jnp.float32(8193.0)
    hi = c - (c - xn)
    mg = jnp.float32(0.75)
    sub = (xn + mg) - mg
    x16 = jnp.where(jnp.abs(xn) < jnp.float32(6.103515625e-05), sub, hi)
    cap = lax.dot_general(x16, lwt_ref[...], (((1,), (0,)), ((), ())),
                          precision=HI, preferred_element_type=F32)
    out_ref[...] = out + cap + lb_ref[...]


def kernel(features, text, atten, linear_W, linear_b, mlp_W1, mlp_b1,
           bn_gamma, bn_beta, mlp_W2, mlp_b2):
    B, L, D = features.shape
    DE = linear_W.shape[0]
    H = mlp_W1.shape[0]
    K = (atten.shape[1] - 2) * 3 // 10
    KP = 160
    N = B * K

    am8 = pl.pallas_call(
        _amax_body,
        out_shape=jax.ShapeDtypeStruct((B, 8), jnp.int32),
        in_specs=[pl.BlockSpec((B, L), lambda: (0, 0))],
        out_specs=pl.BlockSpec((B, 8), lambda: (0, 0)),
    )(text)
    am = am8[:, 0]

    text3 = text.reshape(B, 1, L)
    grid_spec = pltpu.PrefetchScalarGridSpec(
        num_scalar_prefetch=1,
        grid=(B,),
        in_specs=[
            pl.BlockSpec((1, 1, L), lambda b, am_r: (b, 0, 0)),
            pl.BlockSpec((1, 1, L), lambda b, am_r: (b * L + am_r[b], 0, 0)),
            pl.BlockSpec((1, L, D), lambda b, am_r: (b, 0, 0)),
        ],
        out_specs=[
            pl.BlockSpec((1, K, D), lambda b, am_r: (b, 0, 0)),
            pl.BlockSpec((1, 1, KP), lambda b, am_r: (b, 0, 0)),
        ],
    )
    feats, gid = pl.pallas_call(
        functools.partial(_select_body, L=L, KP=KP, K=K),
        grid_spec=grid_spec,
        out_shape=[
            jax.ShapeDtypeStruct((B, K, D), F32),
            jax.ShapeDtypeStruct((B, 1, KP), jnp.int32),
        ],
    )(am, text3, atten.reshape(B * L, 1, L), features)
    del gid

    xflat = feats.reshape(N, D)
    w1t = mlp_W1.T
    w2t = mlp_W2.T
    lwt = linear_W.T
    b1r = mlp_b1.reshape(1, H)
    b2r = mlp_b2.reshape(1, DE)
    lbr = linear_b.reshape(1, DE)
    gr = bn_gamma.reshape(1, H)
    br = bn_beta.reshape(1, H)

    RB = 8 * K
    n_steps = N // RB

    stats = pl.pallas_call(
        _stats_body,
        grid=(n_steps,),
        in_specs=[
            pl.BlockSpec((RB, D), lambda i: (i, 0)),
            pl.BlockSpec((D, H), lambda i: (0, 0)),
            pl.BlockSpec((1, H), lambda i: (0, 0)),
        ],
        out_specs=pl.BlockSpec((8, H), lambda i: (0, 0)),
        out_shape=jax.ShapeDtypeStruct((8, H), F32),
    )(xflat, w1t, b1r)

    res = pl.pallas_call(
        functools.partial(_final_body, n_rows=N),
        grid=(n_steps,),
        in_specs=[
            pl.BlockSpec((RB, D), lambda i: (i, 0)),
            pl.BlockSpec((D, H), lambda i: (0, 0)),
            pl.BlockSpec((1, H), lambda i: (0, 0)),
            pl.BlockSpec((8, H), lambda i: (0, 0)),
            pl.BlockSpec((1, H), lambda i: (0, 0)),
            pl.BlockSpec((1, H), lambda i: (0, 0)),
            pl.BlockSpec((H, DE), lambda i: (0, 0)),
            pl.BlockSpec((1, DE), lambda i: (0, 0)),
            pl.BlockSpec((D, DE), lambda i: (0, 0)),
            pl.BlockSpec((1, DE), lambda i: (0, 0)),
        ],
        out_specs=pl.BlockSpec((RB, DE), lambda i: (i, 0)),
        out_shape=jax.ShapeDtypeStruct((N, DE), F32),
    )(xflat, w1t, b1r, stats, gr, br, w2t, b2r, lwt, lbr)

    return res.reshape(B, K, DE)


# speedup vs baseline: 2.6018x; 2.6018x over previous
import functools

import jax
import jax.numpy as jnp
from jax import lax
from jax.experimental import pallas as pl
from jax.experimental.pallas import tpu as pltpu

HI = lax.Precision.HIGHEST
F32 = jnp.float32
DN_NT = (((1,), (1,)), ((), ()))


def _amax_body(t_ref, am_ref):
    t = t_ref[...]
    B, L = t.shape
    mx = jnp.max(t, axis=1, keepdims=True)
    col = lax.broadcasted_iota(jnp.int32, (B, L), 1)
    cand = jnp.where(t == mx, col, L)
    am = jnp.min(cand, axis=1, keepdims=True)
    am_ref[...] = jnp.broadcast_to(am, (B, 8))


def _select_body(am_ref, t_ref, a_ref, f_ref, w1_ref, b1_ref,
                 xn_ref, gid_ref, st_ref, *, L, KP, K):
    b = pl.program_id(0)
    amb = am_ref[b]
    t = t_ref[0]
    rows = a_ref[0]
    sub = lax.broadcasted_iota(jnp.int32, (8, L), 0)
    a = jnp.sum(jnp.where(sub == amb % 8, rows, 0.0), axis=0, keepdims=True)
    col1 = lax.broadcasted_iota(jnp.int32, (1, L), 1)
    a = jnp.where(col1 == amb, jnp.float32(-1.0), a)
    a = jnp.where(t != 0, a, jnp.float32(0.0))

    A = jnp.broadcast_to(a, (L, L))
    AT = lax.transpose(A, (1, 0))
    J = lax.broadcasted_iota(jnp.int32, (L, L), 1)
    I = lax.broadcasted_iota(jnp.int32, (L, L), 0)
    beat = (A > AT) | ((A == AT) & (J < I))
    rank = jnp.sum(beat.astype(F32), axis=1, keepdims=True)

    ranki = rank.astype(jnp.int32)
    R = lax.broadcasted_iota(jnp.int32, (L, KP), 1)
    O = (ranki == R).astype(F32)
    OT = lax.transpose(O, (1, 0))
    feats = lax.dot_general(OT, f_ref[0], (((1,), (0,)), ((), ())),
                            precision=HI, preferred_element_type=F32)
    feats = feats[:K, :]
    nrm = jnp.sqrt(jnp.sum(feats * feats, axis=1, keepdims=True)) + 1e-8
    xn = feats / nrm
    xn_ref[0] = xn

    Irow = lax.broadcasted_iota(jnp.int32, (L, KP), 0).astype(F32)
    idxf = jnp.sum(O * Irow, axis=0, keepdims=True)
    gid_ref[0] = idxf.astype(jnp.int32) + b * L

    @pl.when(b == 0)
    def _():
        st_ref[...] = jnp.zeros_like(st_ref)

    h = lax.dot_general(xn, w1_ref[...], DN_NT, preferred_element_type=F32)
    h = h + b1_ref[...]
    st_ref[0:1, :] += jnp.sum(h, axis=0, keepdims=True)
    st_ref[1:2, :] += jnp.sum(h * h, axis=0, keepdims=True)


def _final_body(x_ref, w1_ref, b1_ref, st_ref, g_ref, bb_ref, w2_ref,
                b2_ref, lw_ref, lb_ref, out_ref, *, n_rows):
    xn = x_ref[...]
    h = lax.dot_general(xn, w1_ref[...], DN_NT, preferred_element_type=F32)
    h = h + b1_ref[...]
    inv_n = jnp.float32(1.0 / n_rows)
    mean = st_ref[0:1, :] * inv_n
    var = st_ref[1:2, :] * inv_n - mean * mean
    hn = (h - mean) / jnp.sqrt(var + 1e-5) * g_ref[...] + bb_ref[...]
    hr = jnp.maximum(hn, 0.0)
    out = lax.dot_general(hr, w2_ref[...], DN_NT, preferred_element_type=F32)
    out = out + b2_ref[...]
    c = xn * jnp.float32(8193.0)
    hi = c - (c - xn)
    mg = jnp.float32(0.75)
    sub = (xn + mg) - mg
    x16 = jnp.where(jnp.abs(xn) < jnp.float32(6.103515625e-05), sub, hi)
    cap = lax.dot_general(x16, lw_ref[...], DN_NT, preferred_element_type=F32)
    out_ref[...] = out + cap + lb_ref[...]


def kernel(features, text, atten, linear_W, linear_b, mlp_W1, mlp_b1,
           bn_gamma, bn_beta, mlp_W2, mlp_b2):
    B, L, D = features.shape
    DE = linear_W.shape[0]
    H = mlp_W1.shape[0]
    K = (atten.shape[1] - 2) * 3 // 10
    KP = 160
    N = B * K

    am8 = pl.pallas_call(
        _amax_body,
        out_shape=jax.ShapeDtypeStruct((B, 8), jnp.int32),
        in_specs=[pl.BlockSpec((B, L), lambda: (0, 0))],
        out_specs=pl.BlockSpec((B, 8), lambda: (0, 0)),
    )(text)
    am = am8[:, 0]

    text3 = text.reshape(B, 1, L)
    grid_spec = pltpu.PrefetchScalarGridSpec(
        num_scalar_prefetch=1,
        grid=(B,),
        in_specs=[
            pl.BlockSpec((1, 1, L), lambda b, am_r: (b, 0, 0)),
            pl.BlockSpec((1, 8, L), lambda b, am_r: (b, am_r[b] // 8, 0)),
            pl.BlockSpec((1, L, D), lambda b, am_r: (b, 0, 0)),
            pl.BlockSpec((H, D), lambda b, am_r: (0, 0)),
            pl.BlockSpec((1, H), lambda b, am_r: (0, 0)),
        ],
        out_specs=[
            pl.BlockSpec((1, K, D), lambda b, am_r: (b, 0, 0)),
            pl.BlockSpec((1, 1, KP), lambda b, am_r: (b, 0, 0)),
            pl.BlockSpec((8, H), lambda b, am_r: (0, 0)),
        ],
    )
    xn, gid, stats = pl.pallas_call(
        functools.partial(_select_body, L=L, KP=KP, K=K),
        grid_spec=grid_spec,
        out_shape=[
            jax.ShapeDtypeStruct((B, K, D), F32),
            jax.ShapeDtypeStruct((B, 1, KP), jnp.int32),
            jax.ShapeDtypeStruct((8, H), F32),
        ],
    )(am, text3, atten, features, mlp_W1, mlp_b1.reshape(1, H))
    del gid

    xflat = xn.reshape(N, D)
    RB = 8 * K
    n_steps = N // RB

    res = pl.pallas_call(
        functools.partial(_final_body, n_rows=N),
        grid=(n_steps,),
        in_specs=[
            pl.BlockSpec((RB, D), lambda i: (i, 0)),
            pl.BlockSpec((H, D), lambda i: (0, 0)),
            pl.BlockSpec((1, H), lambda i: (0, 0)),
            pl.BlockSpec((8, H), lambda i: (0, 0)),
            pl.BlockSpec((1, H), lambda i: (0, 0)),
            pl.BlockSpec((1, H), lambda i: (0, 0)),
            pl.BlockSpec((DE, H), lambda i: (0, 0)),
            pl.BlockSpec((1, DE), lambda i: (0, 0)),
            pl.BlockSpec((DE, D), lambda i: (0, 0)),
            pl.BlockSpec((1, DE), lambda i: (0, 0)),
        ],
        out_specs=pl.BlockSpec((RB, DE), lambda i: (i, 0)),
        out_shape=jax.ShapeDtypeStruct((N, DE), F32),
    )(xflat, mlp_W1, mlp_b1.reshape(1, H), stats, bn_gamma.reshape(1, H),
      bn_beta.reshape(1, H), mlp_W2, mlp_b2.reshape(1, DE), linear_W,
      linear_b.reshape(1, DE))

    return res.reshape(B, K, DE)


# speedup vs baseline: 3.4540x; 1.3276x over previous
import functools

import jax
import jax.numpy as jnp
from jax import lax
from jax.experimental import pallas as pl
from jax.experimental.pallas import tpu as pltpu

HI = lax.Precision.HIGHEST
F32 = jnp.float32
DN_NT = (((1,), (1,)), ((), ()))


def _amax_body(t_ref, am_ref):
    t = t_ref[...]
    B, L = t.shape
    mx = jnp.max(t, axis=1, keepdims=True)
    col = lax.broadcasted_iota(jnp.int32, (B, L), 1)
    cand = jnp.where(t == mx, col, L)
    am = jnp.min(cand, axis=1, keepdims=True)
    am_ref[...] = jnp.broadcast_to(am, (B, 8))


def _select_body(am_ref, t_ref, a_ref, f_ref, w1_ref, b1_ref,
                 xn_ref, gid_ref, st_ref, *, L, KP, K):
    b = pl.program_id(0)
    amb = am_ref[b]
    t = t_ref[0]
    rows = a_ref[0]
    sub = lax.broadcasted_iota(jnp.int32, (8, L), 0)
    a = jnp.sum(jnp.where(sub == amb % 8, rows, 0.0), axis=0, keepdims=True)
    col1 = lax.broadcasted_iota(jnp.int32, (1, L), 1)
    a = jnp.where(col1 == amb, jnp.float32(-1.0), a)
    a = jnp.where(t != 0, a, jnp.float32(0.0))

    A = jnp.broadcast_to(a, (L, L))
    AT = lax.transpose(A, (1, 0))
    J = lax.broadcasted_iota(jnp.int32, (L, L), 1)
    I = lax.broadcasted_iota(jnp.int32, (L, L), 0)
    beat = (A > AT) | ((A == AT) & (J < I))
    rank = jnp.sum(beat.astype(F32), axis=1, keepdims=True)

    ranki = rank.astype(jnp.int32)
    R = lax.broadcasted_iota(jnp.int32, (L, KP), 1)
    O = (ranki == R).astype(F32)
    OT = lax.transpose(O, (1, 0))
    feats = lax.dot_general(OT, f_ref[0], (((1,), (0,)), ((), ())),
                            precision=HI, preferred_element_type=F32)
    nrm = jnp.sqrt(jnp.sum(feats * feats, axis=1, keepdims=True)) + 1e-8
    xn = feats / nrm
    xn_ref[0] = xn

    Irow = lax.broadcasted_iota(jnp.int32, (L, KP), 0).astype(F32)
    idxf = jnp.sum(O * Irow, axis=0, keepdims=True)
    gid_ref[0] = idxf.astype(jnp.int32) + b * L

    @pl.when(b == 0)
    def _():
        st_ref[...] = jnp.zeros_like(st_ref)

    h = lax.dot_general(xn[:K, :], w1_ref[...], DN_NT,
                        preferred_element_type=F32)
    h = h + b1_ref[...]
    st_ref[0:1, :] += jnp.sum(h, axis=0, keepdims=True)
    st_ref[1:2, :] += jnp.sum(h * h, axis=0, keepdims=True)


def _final_body(x_ref, w1_ref, b1_ref, st_ref, g_ref, bb_ref, w2_ref,
                b2_ref, lw_ref, lb_ref, out_ref, *, n_rows, K, KP):
    xn = x_ref[...]
    h = lax.dot_general(xn, w1_ref[...], DN_NT, preferred_element_type=F32)
    h = h + b1_ref[...]
    inv_n = jnp.float32(1.0 / n_rows)
    mean = st_ref[0:1, :] * inv_n
    var = st_ref[1:2, :] * inv_n - mean * mean
    hn = (h - mean) / jnp.sqrt(var + 1e-5) * g_ref[...] + bb_ref[...]
    hr = jnp.maximum(hn, 0.0)
    out = lax.dot_general(hr, w2_ref[...], DN_NT, preferred_element_type=F32)
    out = out + b2_ref[...]
    c = xn * jnp.float32(8193.0)
    hi = c - (c - xn)
    mg = jnp.float32(0.75)
    sub = (xn + mg) - mg
    x16 = jnp.where(jnp.abs(xn) < jnp.float32(6.103515625e-05), sub, hi)
    cap = lax.dot_general(x16, lw_ref[...], DN_NT, preferred_element_type=F32)
    res = out + cap + lb_ref[...]
    for j in range(8):
        out_ref[j] = res[j * KP:j * KP + K, :]


def kernel(features, text, atten, linear_W, linear_b, mlp_W1, mlp_b1,
           bn_gamma, bn_beta, mlp_W2, mlp_b2):
    B, L, D = features.shape
    DE = linear_W.shape[0]
    H = mlp_W1.shape[0]
    K = (atten.shape[1] - 2) * 3 // 10
    KP = 160
    N = B * K

    am8 = pl.pallas_call(
        _amax_body,
        out_shape=jax.ShapeDtypeStruct((B, 8), jnp.int32),
        in_specs=[pl.BlockSpec((B, L), lambda: (0, 0))],
        out_specs=pl.BlockSpec((B, 8), lambda: (0, 0)),
    )(text)
    am = am8[:, 0]

    text3 = text.reshape(B, 1, L)
    grid_spec = pltpu.PrefetchScalarGridSpec(
        num_scalar_prefetch=1,
        grid=(B,),
        in_specs=[
            pl.BlockSpec((1, 1, L), lambda b, am_r: (b, 0, 0)),
            pl.BlockSpec((1, 8, L), lambda b, am_r: (b, am_r[b] // 8, 0)),
            pl.BlockSpec((1, L, D), lambda b, am_r: (b, 0, 0)),
            pl.BlockSpec((H, D), lambda b, am_r: (0, 0)),
            pl.BlockSpec((1, H), lambda b, am_r: (0, 0)),
        ],
        out_specs=[
            pl.BlockSpec((1, KP, D), lambda b, am_r: (b, 0, 0)),
            pl.BlockSpec((1, 1, KP), lambda b, am_r: (b, 0, 0)),
            pl.BlockSpec((8, H), lambda b, am_r: (0, 0)),
        ],
    )
    xn, gid, stats = pl.pallas_call(
        functools.partial(_select_body, L=L, KP=KP, K=K),
        grid_spec=grid_spec,
        out_shape=[
            jax.ShapeDtypeStruct((B, KP, D), F32),
            jax.ShapeDtypeStruct((B, 1, KP), jnp.int32),
            jax.ShapeDtypeStruct((8, H), F32),
        ],
    )(am, text3, atten, features, mlp_W1, mlp_b1.reshape(1, H))
    del gid

    xflat = xn.reshape(B * KP, D)
    RB = 8 * KP
    n_steps = B * KP // RB

    res = pl.pallas_call(
        functools.partial(_final_body, n_rows=N, K=K, KP=KP),
        grid=(n_steps,),
        in_specs=[
            pl.BlockSpec((RB, D), lambda i: (i, 0)),
            pl.BlockSpec((H, D), lambda i: (0, 0)),
            pl.BlockSpec((1, H), lambda i: (0, 0)),
            pl.BlockSpec((8, H), lambda i: (0, 0)),
            pl.BlockSpec((1, H), lambda i: (0, 0)),
            pl.BlockSpec((1, H), lambda i: (0, 0)),
            pl.BlockSpec((DE, H), lambda i: (0, 0)),
            pl.BlockSpec((1, DE), lambda i: (0, 0)),
            pl.BlockSpec((DE, D), lambda i: (0, 0)),
            pl.BlockSpec((1, DE), lambda i: (0, 0)),
        ],
        out_specs=pl.BlockSpec((8, K, DE), lambda i: (i, 0, 0)),
        out_shape=jax.ShapeDtypeStruct((B, K, DE), F32),
    )(xflat, mlp_W1, mlp_b1.reshape(1, H), stats, bn_gamma.reshape(1, H),
      bn_beta.reshape(1, H), mlp_W2, mlp_b2.reshape(1, DE), linear_W,
      linear_b.reshape(1, DE))

    return res


# speedup vs baseline: 4.0656x; 1.1771x over previous
import functools

import jax
import jax.numpy as jnp
from jax import lax
from jax.experimental import pallas as pl
from jax.experimental.pallas import tpu as pltpu

HI = lax.Precision.HIGHEST
F32 = jnp.float32
DN_NT = (((1,), (1,)), ((), ()))


def _amax_body(t_ref, am_ref):
    t = t_ref[...]
    B, L = t.shape
    mx = jnp.max(t, axis=1, keepdims=True)
    col = lax.broadcasted_iota(jnp.int32, (B, L), 1)
    cand = jnp.where(t == mx, col, L)
    am = jnp.min(cand, axis=1, keepdims=True)
    am_ref[...] = jnp.broadcast_to(am, (B, 8))


def _select_body(am_ref, t_ref, a_ref, f_ref, w1_ref, b1_ref,
                 xn_ref, gid_ref, st_ref, *, L, KP, K):
    b = pl.program_id(0)
    amb = am_ref[b]
    t = t_ref[0]
    rows = a_ref[0]
    sub = lax.broadcasted_iota(jnp.int32, (8, L), 0)
    a = jnp.sum(jnp.where(sub == amb % 8, rows, 0.0), axis=0, keepdims=True)
    col1 = lax.broadcasted_iota(jnp.int32, (1, L), 1)
    a = jnp.where(col1 == amb, jnp.float32(-1.0), a)
    a = jnp.where(t != 0, a, jnp.float32(0.0))

    A = jnp.broadcast_to(a, (L, L))
    AT = lax.transpose(A, (1, 0))
    J = lax.broadcasted_iota(jnp.int32, (L, L), 1)
    I = lax.broadcasted_iota(jnp.int32, (L, L), 0)
    beat = (A > AT) | ((A == AT) & (J < I))
    rank = jnp.sum(beat.astype(F32), axis=1, keepdims=True)

    ranki = rank.astype(jnp.int32)
    R = lax.broadcasted_iota(jnp.int32, (L, KP), 1)
    O = (ranki == R).astype(F32)
    OT = lax.transpose(O, (1, 0))
    feats = lax.dot_general(OT, f_ref[0], (((1,), (0,)), ((), ())),
                            preferred_element_type=F32)
    nrm = jnp.sqrt(jnp.sum(feats * feats, axis=1, keepdims=True)) + 1e-8
    xn = feats / nrm
    xn_ref[0] = xn

    Irow = lax.broadcasted_iota(jnp.int32, (L, KP), 0).astype(F32)
    idxf = jnp.sum(O * Irow, axis=0, keepdims=True)
    gid_ref[0] = idxf.astype(jnp.int32) + b * L

    @pl.when(b == 0)
    def _():
        st_ref[...] = jnp.zeros_like(st_ref)

    h = lax.dot_general(xn[:K, :], w1_ref[...], DN_NT,
                        preferred_element_type=F32)
    h = h + b1_ref[...]
    st_ref[0:1, :] += jnp.sum(h, axis=0, keepdims=True)
    st_ref[1:2, :] += jnp.sum(h * h, axis=0, keepdims=True)


def _final_body(x_ref, w1_ref, b1_ref, st_ref, g_ref, bb_ref, w2_ref,
                b2_ref, lw_ref, lb_ref, out_ref, *, n_rows, K, KP):
    xn = x_ref[...]
    h = lax.dot_general(xn, w1_ref[...], DN_NT, preferred_element_type=F32)
    h = h + b1_ref[...]
    inv_n = jnp.float32(1.0 / n_rows)
    mean = st_ref[0:1, :] * inv_n
    var = st_ref[1:2, :] * inv_n - mean * mean
    hn = (h - mean) / jnp.sqrt(var + 1e-5) * g_ref[...] + bb_ref[...]
    hr = jnp.maximum(hn, 0.0)
    out = lax.dot_general(hr, w2_ref[...], DN_NT, preferred_element_type=F32)
    out = out + b2_ref[...]
    c = xn * jnp.float32(8193.0)
    hi = c - (c - xn)
    mg = jnp.float32(0.75)
    sub = (xn + mg) - mg
    x16 = jnp.where(jnp.abs(xn) < jnp.float32(6.103515625e-05), sub, hi)
    cap = lax.dot_general(x16, lw_ref[...], DN_NT, preferred_element_type=F32)
    res = out + cap + lb_ref[...]
    for j in range(8):
        out_ref[j] = res[j * KP:j * KP + K, :]


def kernel(features, text, atten, linear_W, linear_b, mlp_W1, mlp_b1,
           bn_gamma, bn_beta, mlp_W2, mlp_b2):
    B, L, D = features.shape
    DE = linear_W.shape[0]
    H = mlp_W1.shape[0]
    K = (atten.shape[1] - 2) * 3 // 10
    KP = 160
    N = B * K

    am8 = pl.pallas_call(
        _amax_body,
        out_shape=jax.ShapeDtypeStruct((B, 8), jnp.int32),
        in_specs=[pl.BlockSpec((B, L), lambda: (0, 0))],
        out_specs=pl.BlockSpec((B, 8), lambda: (0, 0)),
    )(text)
    am = am8[:, 0]

    text3 = text.reshape(B, 1, L)
    grid_spec = pltpu.PrefetchScalarGridSpec(
        num_scalar_prefetch=1,
        grid=(B,),
        in_specs=[
            pl.BlockSpec((1, 1, L), lambda b, am_r: (b, 0, 0)),
            pl.BlockSpec((1, 8, L), lambda b, am_r: (b, am_r[b] // 8, 0)),
            pl.BlockSpec((1, L, D), lambda b, am_r: (b, 0, 0)),
            pl.BlockSpec((H, D), lambda b, am_r: (0, 0)),
            pl.BlockSpec((1, H), lambda b, am_r: (0, 0)),
        ],
        out_specs=[
            pl.BlockSpec((1, KP, D), lambda b, am_r: (b, 0, 0)),
            pl.BlockSpec((1, 1, KP), lambda b, am_r: (b, 0, 0)),
            pl.BlockSpec((8, H), lambda b, am_r: (0, 0)),
        ],
    )
    xn, gid, stats = pl.pallas_call(
        functools.partial(_select_body, L=L, KP=KP, K=K),
        grid_spec=grid_spec,
        out_shape=[
            jax.ShapeDtypeStruct((B, KP, D), F32),
            jax.ShapeDtypeStruct((B, 1, KP), jnp.int32),
            jax.ShapeDtypeStruct((8, H), F32),
        ],
    )(am, text3, atten, features, mlp_W1, mlp_b1.reshape(1, H))
    del gid

    xflat = xn.reshape(B * KP, D)
    RB = 8 * KP
    n_steps = B * KP // RB

    res = pl.pallas_call(
        functools.partial(_final_body, n_rows=N, K=K, KP=KP),
        grid=(n_steps,),
        in_specs=[
            pl.BlockSpec((RB, D), lambda i: (i, 0)),
            pl.BlockSpec((H, D), lambda i: (0, 0)),
            pl.BlockSpec((1, H), lambda i: (0, 0)),
            pl.BlockSpec((8, H), lambda i: (0, 0)),
            pl.BlockSpec((1, H), lambda i: (0, 0)),
            pl.BlockSpec((1, H), lambda i: (0, 0)),
            pl.BlockSpec((DE, H), lambda i: (0, 0)),
            pl.BlockSpec((1, DE), lambda i: (0, 0)),
            pl.BlockSpec((DE, D), lambda i: (0, 0)),
            pl.BlockSpec((1, DE), lambda i: (0, 0)),
        ],
        out_specs=pl.BlockSpec((8, K, DE), lambda i: (i, 0, 0)),
        out_shape=jax.ShapeDtypeStruct((B, K, DE), F32),
    )(xflat, mlp_W1, mlp_b1.reshape(1, H), stats, bn_gamma.reshape(1, H),
      bn_beta.reshape(1, H), mlp_W2, mlp_b2.reshape(1, DE), linear_W,
      linear_b.reshape(1, DE))

    return res
